# baseline (speedup 1.0000x reference)
import jax
import jax.numpy as jnp
from jax import lax
from jax.experimental import pallas as pl
from jax.experimental.pallas import tpu as pltpu
from jax.experimental.pallas import tpu_sc as plsc

N = 10000
E = 320000
D = 128
H = 128
OUT = 64
G = 128
EPS = 0.0
BNEPS = 1e-5

NC = 2
NS = 16
NW = NC * NS
K = 96
C = 112
CAP = NW * C * K
RPT = 632
N_ACC = NS * RPT
N_OUT = N_ACC
CH = 16

BR = 400
NB = N // BR


NBUF = 3


def _seg_sum_body(tbl_hbm, srcs_hbm, dsts_hbm, zeros_hbm, out_hbm,
                  src_v, dst_v, acc, *bufs_sems):
    rows = bufs_sems[:NBUF]
    gsem = bufs_sems[NBUF:2 * NBUF]
    ssem = bufs_sems[2 * NBUF:]
    c = lax.axis_index("c")
    s = lax.axis_index("s")
    wid = s * NC + c

    pltpu.sync_copy(zeros_hbm, acc.at[pl.ds(s * RPT, RPT)])
    plsc.subcore_barrier()

    for h in range(C // CH):
        pltpu.sync_copy(srcs_hbm.at[wid, pl.ds(h * CH, CH)], src_v)
        pltpu.sync_copy(dsts_hbm.at[wid, pl.ds(h * CH, CH)], dst_v)
        for b in range(NBUF):
            pltpu.async_copy(tbl_hbm.at[src_v.at[b]], rows[b], gsem[b])

        def outer(j, carry):
            for b in range(NBUF):
                e = j * NBUF + b
                pltpu.make_async_copy(tbl_hbm.at[src_v.at[e]], rows[b],
                                      gsem[b]).wait()
                pltpu.async_copy(rows[b], acc.at[dst_v.at[e]], ssem[b],
                                 add=True)
                en = e + NBUF

                @pl.when(en < CH)
                def _():
                    pltpu.make_async_copy(rows[b], acc.at[dst_v.at[e]],
                                          ssem[b]).wait()
                    pltpu.async_copy(tbl_hbm.at[src_v.at[en]], rows[b],
                                     gsem[b])
            return carry

        nfull = (CH // NBUF) - 1
        lax.fori_loop(0, nfull, outer, 0)
        for e in range(nfull * NBUF, CH):
            b = e % NBUF
            pltpu.make_async_copy(tbl_hbm.at[src_v.at[e]], rows[b],
                                  gsem[b]).wait()
            pltpu.async_copy(rows[b], acc.at[dst_v.at[e]], ssem[b],
                             add=True)
            en = e + NBUF
            if en < CH:
                pltpu.make_async_copy(rows[b], acc.at[dst_v.at[e]],
                                      ssem[b]).wait()
                pltpu.async_copy(tbl_hbm.at[src_v.at[en]], rows[b],
                                 gsem[b])
        for e in range(CH - NBUF, CH):
            b = e % NBUF
            pltpu.make_async_copy(rows[b], acc.at[dst_v.at[e]],
                                  ssem[b]).wait()

    plsc.subcore_barrier()
    pltpu.sync_copy(acc.at[pl.ds(s * RPT, RPT)],
                    out_hbm.at[c, pl.ds(s * RPT, RPT)])


def _seg_sum_call(table, srcs3, dsts3, zeros_z):
    mesh = plsc.VectorSubcoreMesh(core_axis_name="c", subcore_axis_name="s",
                                  num_cores=NC, num_subcores=NS)
    return pl.kernel(
        _seg_sum_body,
        out_type=jax.ShapeDtypeStruct((NC, N_OUT, D), jnp.float32),
        mesh=mesh,
        scratch_types=[
            pltpu.VMEM((CH, K), jnp.int32),
            pltpu.VMEM((CH, K), jnp.int32),
            pltpu.VMEM_SHARED((N_ACC, D), jnp.float32),
        ] + [pltpu.VMEM((K, D), jnp.float32)] * NBUF
          + [pltpu.SemaphoreType.DMA] * (2 * NBUF),
    )(table, srcs3, dsts3, zeros_z)


def _mlp_bn_body(x_ref, agg_ref, wa_ref, ba_ref, wb_ref, bb_ref, g_ref,
                 be_ref, o_ref, hbuf, s_ref, q_ref):
    p = pl.program_id(0)
    i = pl.program_id(1)

    @pl.when(p == 0)
    def _():
        z = x_ref[...] * (1.0 + EPS) + agg_ref[0] + agg_ref[1]
        a = jnp.maximum(
            jnp.dot(z, wa_ref[...], preferred_element_type=jnp.float32)
            + ba_ref[...], 0.0)
        h = (jnp.dot(a, wb_ref[...], preferred_element_type=jnp.float32)
             + bb_ref[...])
        hbuf[pl.ds(i * BR, BR), :] = h

        @pl.when(i == 0)
        def _():
            s_ref[...] = jnp.zeros_like(s_ref)
            q_ref[...] = jnp.zeros_like(q_ref)

        s_ref[...] += jnp.sum(h, axis=0, keepdims=True)
        q_ref[...] += jnp.sum(h * h, axis=0, keepdims=True)

    @pl.when(p == 1)
    def _():
        mu = s_ref[...] * (1.0 / N)
        var = q_ref[...] * (1.0 / N) - mu * mu
        inv = g_ref[...] * lax.rsqrt(var + BNEPS)
        sh = be_ref[...] - mu * inv
        o_ref[...] = jnp.maximum(hbuf[pl.ds(i * BR, BR), :] * inv + sh, 0.0)


def _mlp_bn_call(xin, aggp, wa, ba, wb, bb, g, be):
    const = pl.BlockSpec((1, H), lambda p, i: (0, 0))
    return pl.pallas_call(
        _mlp_bn_body,
        grid=(2, NB),
        in_specs=[
            pl.BlockSpec((BR, D), lambda p, i: (i * (1 - p), 0)),
            pl.BlockSpec((NC, BR, D), lambda p, i: (0, i * (1 - p), 0)),
            pl.BlockSpec((D, H), lambda p, i: (0, 0)),
            const,
            pl.BlockSpec((H, H), lambda p, i: (0, 0)),
            const, const, const,
        ],
        out_specs=pl.BlockSpec((BR, H), lambda p, i: (i * p, 0)),
        out_shape=jax.ShapeDtypeStruct((N, H), jnp.float32),
        scratch_shapes=[
            pltpu.VMEM((N, H), jnp.float32),
            pltpu.VMEM((1, H), jnp.float32),
            pltpu.VMEM((1, H), jnp.float32),
        ],
    )(xin, aggp, wa, ba, wb, bb, g, be)


def _mlp_bn_pool_body(x_ref, agg_ref, wa_ref, ba_ref, wb_ref, bb_ref, g_ref,
                      be_ref, bat_ref, wl_ref, bl_ref, o_ref,
                      hbuf, s_ref, q_ref, pool_ref):
    p = pl.program_id(0)
    i = pl.program_id(1)

    @pl.when(p == 0)
    def _():
        z = x_ref[...] * (1.0 + EPS) + agg_ref[0] + agg_ref[1]
        a = jnp.maximum(
            jnp.dot(z, wa_ref[...], preferred_element_type=jnp.float32)
            + ba_ref[...], 0.0)
        h = (jnp.dot(a, wb_ref[...], preferred_element_type=jnp.float32)
             + bb_ref[...])
        hbuf[pl.ds(i * BR, BR), :] = h

        @pl.when(i == 0)
        def _():
            s_ref[...] = jnp.zeros_like(s_ref)
            q_ref[...] = jnp.zeros_like(q_ref)

        s_ref[...] += jnp.sum(h, axis=0, keepdims=True)
        q_ref[...] += jnp.sum(h * h, axis=0, keepdims=True)

    @pl.when(p == 1)
    def _():
        mu = s_ref[...] * (1.0 / N)
        var = q_ref[...] * (1.0 / N) - mu * mu
        inv = g_ref[...] * lax.rsqrt(var + BNEPS)
        sh = be_ref[...] - mu * inv
        h = jnp.maximum(hbuf[pl.ds(i * BR, BR), :] * inv + sh, 0.0)

        bat = bat_ref[0]
        onehot = (lax.broadcasted_iota(jnp.int32, (G, BR), 0) == bat
                  ).astype(jnp.float32)
        part = jnp.dot(onehot, h, preferred_element_type=jnp.float32)

        @pl.when(i == 0)
        def _():
            pool_ref[...] = jnp.zeros_like(pool_ref)

        pool_ref[...] += part

        @pl.when(i == NB - 1)
        def _():
            logits = (jnp.dot(pool_ref[...], wl_ref[...],
                              preferred_element_type=jnp.float32)
                      + bl_ref[...])
            m = jnp.max(logits, axis=1, keepdims=True)
            lse = jnp.log(jnp.sum(jnp.exp(logits - m), axis=1, keepdims=True))
            o_ref[...] = logits - m - lse


def _mlp_bn_pool_call(xin, aggp, wa, ba, wb, bb, g, be, bat3, wl, bl):
    const = pl.BlockSpec((1, H), lambda p, i: (0, 0))
    return pl.pallas_call(
        _mlp_bn_pool_body,
        grid=(2, NB),
        in_specs=[
            pl.BlockSpec((BR, D), lambda p, i: (i * (1 - p), 0)),
            pl.BlockSpec((NC, BR, D), lambda p, i: (0, i * (1 - p), 0)),
            pl.BlockSpec((D, H), lambda p, i: (0, 0)),
            const,
            pl.BlockSpec((H, H), lambda p, i: (0, 0)),
            const, const, const,
            pl.BlockSpec((1, 1, BR), lambda p, i: (i * p, 0, 0)),
            pl.BlockSpec((H, OUT), lambda p, i: (0, 0)),
            pl.BlockSpec((1, OUT), lambda p, i: (0, 0)),
        ],
        out_specs=pl.BlockSpec((G, OUT), lambda p, i: (0, 0)),
        out_shape=jax.ShapeDtypeStruct((G, OUT), jnp.float32),
        scratch_shapes=[
            pltpu.VMEM((N, H), jnp.float32),
            pltpu.VMEM((1, H), jnp.float32),
            pltpu.VMEM((1, H), jnp.float32),
            pltpu.VMEM((G, H), jnp.float32),
        ],
    )(xin, aggp, wa, ba, wb, bb, g, be, bat3, wl, bl)


def kernel(x, edge_index, batch, W1a, b1a, W1b, b1b, g1, be1,
           W2a, b2a, W2b, b2b, g2, be2, Wl, bl):
    src = edge_index[0]
    dst = edge_index[1]
    pad = CAP - E
    dst_pad = N + (jnp.arange(pad, dtype=jnp.int32) % (N_ACC - N))
    srcs3 = jnp.concatenate(
        [src, jnp.zeros((pad,), jnp.int32)]).reshape(C, NW, K).transpose(1, 0, 2)
    dsts3 = jnp.concatenate(
        [dst, dst_pad]).reshape(C, NW, K).transpose(1, 0, 2)
    zeros_z = jnp.zeros((RPT, D), jnp.float32)
    bat3 = batch.reshape(NB, 1, BR)

    b1a2, b1b2 = b1a.reshape(1, H), b1b.reshape(1, H)
    b2a2, b2b2 = b2a.reshape(1, H), b2b.reshape(1, H)
    g12, be12 = g1.reshape(1, H), be1.reshape(1, H)
    g22, be22 = g2.reshape(1, H), be2.reshape(1, H)
    bl2 = bl.reshape(1, OUT)

    agg1 = _seg_sum_call(x, srcs3, dsts3, zeros_z)
    hn1 = _mlp_bn_call(x, agg1, W1a, b1a2, W1b, b1b2, g12, be12)

    agg2 = _seg_sum_call(hn1, srcs3, dsts3, zeros_z)
    return _mlp_bn_pool_call(hn1, agg2, W2a, b2a2, W2b, b2b2, g22, be22,
                             bat3, Wl, bl2)


# speedup vs baseline: 2.4077x; 2.4077x over previous
import jax
import jax.numpy as jnp
from jax import lax
from jax.experimental import pallas as pl
from jax.experimental.pallas import tpu as pltpu
from jax.experimental.pallas import tpu_sc as plsc

N = 10000
E = 320000
D = 128
H = 128
OUT = 64
G = 128
EPS = 0.0
BNEPS = 1e-5

NC = 2
NS = 16
NW = NC * NS
K = 128
C = 80
CAP = NW * C * K
RPT = 632
N_ACC = NS * RPT
N_OUT = N_ACC
CH = 40

BR = 400
NB = N // BR


NBUF = 2


def _seg_sum_body(tbl_hbm, srcs_hbm, dsts_hbm, zeros_hbm, out_hbm,
                  src_v, dst_v, acc, *bufs_sems):
    rows = bufs_sems[:NBUF]
    gsem = bufs_sems[NBUF:2 * NBUF]
    ssem = bufs_sems[2 * NBUF:]
    c = lax.axis_index("c")
    s = lax.axis_index("s")
    wid = s * NC + c

    pltpu.sync_copy(zeros_hbm, acc.at[pl.ds(s * RPT, RPT)])
    plsc.subcore_barrier()

    for h in range(C // CH):
        pltpu.sync_copy(srcs_hbm.at[wid, pl.ds(h * CH, CH)], src_v)
        pltpu.sync_copy(dsts_hbm.at[wid, pl.ds(h * CH, CH)], dst_v)
        for b in range(NBUF):
            pltpu.async_copy(tbl_hbm.at[src_v.at[b]], rows[b], gsem[b])

        def outer(j, carry):
            for b in range(NBUF):
                e = j * NBUF + b
                pltpu.make_async_copy(tbl_hbm.at[src_v.at[e]], rows[b],
                                      gsem[b]).wait()
                pltpu.async_copy(rows[b], acc.at[dst_v.at[e]], ssem[b],
                                 add=True)
                en = e + NBUF

                @pl.when(en < CH)
                def _():
                    pltpu.make_async_copy(rows[b], acc.at[dst_v.at[e]],
                                          ssem[b]).wait()
                    pltpu.async_copy(tbl_hbm.at[src_v.at[en]], rows[b],
                                     gsem[b])
            return carry

        lax.fori_loop(0, CH // NBUF, outer, 0)
        for b in range(NBUF):
            pltpu.make_async_copy(rows[b], acc.at[dst_v.at[CH - NBUF + b]],
                                  ssem[b]).wait()

    plsc.subcore_barrier()
    pltpu.sync_copy(acc.at[pl.ds(s * RPT, RPT)],
                    out_hbm.at[c, pl.ds(s * RPT, RPT)])


def _seg_sum_call(table, srcs3, dsts3, zeros_z):
    mesh = plsc.VectorSubcoreMesh(core_axis_name="c", subcore_axis_name="s",
                                  num_cores=NC, num_subcores=NS)
    return pl.kernel(
        _seg_sum_body,
        out_type=jax.ShapeDtypeStruct((NC, N_OUT, D), jnp.float32),
        mesh=mesh,
        scratch_types=[
            pltpu.VMEM((CH, K), jnp.int32),
            pltpu.VMEM((CH, K), jnp.int32),
            pltpu.VMEM_SHARED((N_ACC, D), jnp.float32),
        ] + [pltpu.VMEM((K, D), jnp.float32)] * NBUF
          + [pltpu.SemaphoreType.DMA] * (2 * NBUF),
    )(table, srcs3, dsts3, zeros_z)


def _mlp_bn_body(x_ref, agg_ref, wa_ref, ba_ref, wb_ref, bb_ref, g_ref,
                 be_ref, o_ref, hbuf, s_ref, q_ref):
    p = pl.program_id(0)
    i = pl.program_id(1)

    @pl.when(p == 0)
    def _():
        z = x_ref[...] * (1.0 + EPS) + agg_ref[0] + agg_ref[1]
        a = jnp.maximum(
            jnp.dot(z, wa_ref[...], preferred_element_type=jnp.float32)
            + ba_ref[...], 0.0)
        h = (jnp.dot(a, wb_ref[...], preferred_element_type=jnp.float32)
             + bb_ref[...])
        hbuf[pl.ds(i * BR, BR), :] = h

        @pl.when(i == 0)
        def _():
            s_ref[...] = jnp.zeros_like(s_ref)
            q_ref[...] = jnp.zeros_like(q_ref)

        s_ref[...] += jnp.sum(h, axis=0, keepdims=True)
        q_ref[...] += jnp.sum(h * h, axis=0, keepdims=True)

    @pl.when(p == 1)
    def _():
        mu = s_ref[...] * (1.0 / N)
        var = q_ref[...] * (1.0 / N) - mu * mu
        inv = g_ref[...] * lax.rsqrt(var + BNEPS)
        sh = be_ref[...] - mu * inv
        o_ref[...] = jnp.maximum(hbuf[pl.ds(i * BR, BR), :] * inv + sh, 0.0)


def _mlp_bn_call(xin, aggp, wa, ba, wb, bb, g, be):
    const = pl.BlockSpec((1, H), lambda p, i: (0, 0))
    return pl.pallas_call(
        _mlp_bn_body,
        grid=(2, NB),
        in_specs=[
            pl.BlockSpec((BR, D), lambda p, i: (i * (1 - p), 0)),
            pl.BlockSpec((NC, BR, D), lambda p, i: (0, i * (1 - p), 0)),
            pl.BlockSpec((D, H), lambda p, i: (0, 0)),
            const,
            pl.BlockSpec((H, H), lambda p, i: (0, 0)),
            const, const, const,
        ],
        out_specs=pl.BlockSpec((BR, H), lambda p, i: (i * p, 0)),
        out_shape=jax.ShapeDtypeStruct((N, H), jnp.float32),
        scratch_shapes=[
            pltpu.VMEM((N, H), jnp.float32),
            pltpu.VMEM((1, H), jnp.float32),
            pltpu.VMEM((1, H), jnp.float32),
        ],
    )(xin, aggp, wa, ba, wb, bb, g, be)


def _mlp_bn_pool_body(x_ref, agg_ref, wa_ref, ba_ref, wb_ref, bb_ref, g_ref,
                      be_ref, bat_ref, wl_ref, bl_ref, o_ref,
                      hbuf, s_ref, q_ref, pool_ref):
    p = pl.program_id(0)
    i = pl.program_id(1)

    @pl.when(p == 0)
    def _():
        z = x_ref[...] * (1.0 + EPS) + agg_ref[0] + agg_ref[1]
        a = jnp.maximum(
            jnp.dot(z, wa_ref[...], preferred_element_type=jnp.float32)
            + ba_ref[...], 0.0)
        h = (jnp.dot(a, wb_ref[...], preferred_element_type=jnp.float32)
             + bb_ref[...])
        hbuf[pl.ds(i * BR, BR), :] = h

        @pl.when(i == 0)
        def _():
            s_ref[...] = jnp.zeros_like(s_ref)
            q_ref[...] = jnp.zeros_like(q_ref)

        s_ref[...] += jnp.sum(h, axis=0, keepdims=True)
        q_ref[...] += jnp.sum(h * h, axis=0, keepdims=True)

    @pl.when(p == 1)
    def _():
        mu = s_ref[...] * (1.0 / N)
        var = q_ref[...] * (1.0 / N) - mu * mu
        inv = g_ref[...] * lax.rsqrt(var + BNEPS)
        sh = be_ref[...] - mu * inv
        h = jnp.maximum(hbuf[pl.ds(i * BR, BR), :] * inv + sh, 0.0)

        bat = bat_ref[0]
        onehot = (lax.broadcasted_iota(jnp.int32, (G, BR), 0) == bat
                  ).astype(jnp.float32)
        part = jnp.dot(onehot, h, preferred_element_type=jnp.float32)

        @pl.when(i == 0)
        def _():
            pool_ref[...] = jnp.zeros_like(pool_ref)

        pool_ref[...] += part

        @pl.when(i == NB - 1)
        def _():
            logits = (jnp.dot(pool_ref[...], wl_ref[...],
                              preferred_element_type=jnp.float32)
                      + bl_ref[...])
            m = jnp.max(logits, axis=1, keepdims=True)
            lse = jnp.log(jnp.sum(jnp.exp(logits - m), axis=1, keepdims=True))
            o_ref[...] = logits - m - lse


def _mlp_bn_pool_call(xin, aggp, wa, ba, wb, bb, g, be, bat3, wl, bl):
    const = pl.BlockSpec((1, H), lambda p, i: (0, 0))
    return pl.pallas_call(
        _mlp_bn_pool_body,
        grid=(2, NB),
        in_specs=[
            pl.BlockSpec((BR, D), lambda p, i: (i * (1 - p), 0)),
            pl.BlockSpec((NC, BR, D), lambda p, i: (0, i * (1 - p), 0)),
            pl.BlockSpec((D, H), lambda p, i: (0, 0)),
            const,
            pl.BlockSpec((H, H), lambda p, i: (0, 0)),
            const, const, const,
            pl.BlockSpec((1, 1, BR), lambda p, i: (i * p, 0, 0)),
            pl.BlockSpec((H, OUT), lambda p, i: (0, 0)),
            pl.BlockSpec((1, OUT), lambda p, i: (0, 0)),
        ],
        out_specs=pl.BlockSpec((G, OUT), lambda p, i: (0, 0)),
        out_shape=jax.ShapeDtypeStruct((G, OUT), jnp.float32),
        scratch_shapes=[
            pltpu.VMEM((N, H), jnp.float32),
            pltpu.VMEM((1, H), jnp.float32),
            pltpu.VMEM((1, H), jnp.float32),
            pltpu.VMEM((G, H), jnp.float32),
        ],
    )(xin, aggp, wa, ba, wb, bb, g, be, bat3, wl, bl)


def kernel(x, edge_index, batch, W1a, b1a, W1b, b1b, g1, be1,
           W2a, b2a, W2b, b2b, g2, be2, Wl, bl):
    src = edge_index[0]
    dst = edge_index[1]
    pad = CAP - E
    dst_pad = N + (jnp.arange(pad, dtype=jnp.int32) % (N_ACC - N))
    srcs3 = jnp.concatenate(
        [src, jnp.zeros((pad,), jnp.int32)]).reshape(C, NW, K).transpose(1, 0, 2)
    dsts3 = jnp.concatenate(
        [dst, dst_pad]).reshape(C, NW, K).transpose(1, 0, 2)
    zeros_z = jnp.zeros((RPT, D), jnp.float32)
    bat3 = batch.reshape(NB, 1, BR)

    b1a2, b1b2 = b1a.reshape(1, H), b1b.reshape(1, H)
    b2a2, b2b2 = b2a.reshape(1, H), b2b.reshape(1, H)
    g12, be12 = g1.reshape(1, H), be1.reshape(1, H)
    g22, be22 = g2.reshape(1, H), be2.reshape(1, H)
    bl2 = bl.reshape(1, OUT)

    agg1 = _seg_sum_call(x, srcs3, dsts3, zeros_z)
    hn1 = _mlp_bn_call(x, agg1, W1a, b1a2, W1b, b1b2, g12, be12)

    agg2 = _seg_sum_call(hn1, srcs3, dsts3, zeros_z)
    return _mlp_bn_pool_call(hn1, agg2, W2a, b2a2, W2b, b2b2, g22, be22,
                             bat3, Wl, bl2)
